# Initial kernel scaffold; baseline (speedup 1.0000x reference)
import jax
import jax.numpy as jnp
from jax.experimental import pallas as pl


def kernel(h, o, tfidf, edge_src, edge_dst, W_fc, W_fc1, W_feat, b_feat, W_attn):
    raise NotImplementedError("write your pallas kernel here")


# speedup vs baseline: 4.4862x; 4.4862x over previous
import functools

import jax
import jax.numpy as jnp
from jax import lax
from jax.experimental import pallas as pl
from jax.experimental.pallas import tpu as pltpu
from jax.experimental.pallas import tpu_sc as plsc

N_S = 10000
N_W = 10000
E = 320000
OUT = 128
L = 16
NC = 2
NSUB = 16
NTILE = NC * NSUB
EDGES_PER_TILE = E // NTILE
CHUNK = 80
NCHUNK = EDGES_PER_TILE // CHUNK
ROWS_PER_TILE = 640
ROW_LAST = N_W - ROWS_PER_TILE


def _mm_body(x_ref, w_ref, o_ref):
    o_ref[...] = jnp.dot(x_ref[...], w_ref[...],
                         preferred_element_type=jnp.float32)


def _matmul(x, w, blk):
    m, k = x.shape
    _, n = w.shape
    return pl.pallas_call(
        _mm_body,
        grid=(m // blk,),
        in_specs=[pl.BlockSpec((blk, k), lambda i: (i, 0)),
                  pl.BlockSpec((k, n), lambda i: (0, 0))],
        out_specs=pl.BlockSpec((blk, n), lambda i: (i, 0)),
        out_shape=jax.ShapeDtypeStruct((m, n), jnp.float32),
    )(x, w)


def _mm_bias_body(x_ref, w_ref, b_ref, o_ref):
    o_ref[...] = jnp.dot(x_ref[...], w_ref[...],
                         preferred_element_type=jnp.float32) + b_ref[...]


def _matmul_bias(x, w, b, blk):
    m, k = x.shape
    _, n = w.shape
    return pl.pallas_call(
        _mm_bias_body,
        grid=(m // blk,),
        in_specs=[pl.BlockSpec((blk, k), lambda i: (i, 0)),
                  pl.BlockSpec((k, n), lambda i: (0, 0)),
                  pl.BlockSpec((1, n), lambda i: (0, 0))],
        out_specs=pl.BlockSpec((blk, n), lambda i: (i, 0)),
        out_shape=jax.ShapeDtypeStruct((m, n), jnp.float32),
    )(x, w, b.reshape(1, n))


_MESH = plsc.VectorSubcoreMesh(core_axis_name="c", subcore_axis_name="s")


@functools.partial(
    pl.kernel,
    out_type=(jax.ShapeDtypeStruct((NC, N_W, OUT), jnp.float32),
              jax.ShapeDtypeStruct((NTILE, N_W), jnp.float32)),
    mesh=_MESH,
    compiler_params=pltpu.CompilerParams(needs_layout_passes=False),
    scratch_types=[
        pltpu.VMEM((CHUNK,), jnp.int32),
        pltpu.VMEM((CHUNK,), jnp.int32),
        pltpu.VMEM((CHUNK, OUT), jnp.float32),
        pltpu.VMEM((CHUNK, OUT), jnp.float32),
        pltpu.VMEM((L, OUT), jnp.float32),
        pltpu.VMEM((N_W,), jnp.float32),
        pltpu.VMEM((L * L,), jnp.float32),
        pltpu.VMEM((L,), jnp.float32),
        pltpu.VMEM((OUT,), jnp.float32),
        pltpu.VMEM((CHUNK,), jnp.int32),
        pltpu.VMEM_SHARED((N_W, OUT), jnp.float32),
    ],
)
def _edge_kernel(z_hbm, z1_hbm, dfeat_hbm, src_hbm, dst_hbm, wattn_hbm,
                 zacc_hbm, acc_out, den_out,
                 srcv, dstv, zv, z1v, dfv, denv, redv, eegv, wav, rowsv,
                 acc_sh):
    c = lax.axis_index("c")
    s = lax.axis_index("s")
    wid = s * NC + c
    r0 = pl.multiple_of(jnp.minimum(s * ROWS_PER_TILE, ROW_LAST), 8)

    pltpu.sync_copy(wattn_hbm, wav)

    lane16 = lax.iota(jnp.int32, L)

    def _fill_rows(jb):
        for q in range(CHUNK // L):
            rowsv[pl.ds(q * L, L)] = r0 + jb * CHUNK + q * L + lane16

    pltpu.sync_copy(zacc_hbm, zv)
    for jb in range(8):
        _fill_rows(jb)
        pltpu.sync_copy(zv, acc_sh.at[rowsv])

    zero16 = jnp.zeros((L,), jnp.float32)

    def zinit(q, carry):
        denv[pl.ds(pl.multiple_of(q * L, 8), L)] = zero16
        return carry

    lax.fori_loop(0, N_W // L, zinit, 0)
    plsc.subcore_barrier()

    base0 = wid * EDGES_PER_TILE

    def chunk_body(t, carry):
        base = pl.multiple_of(base0 + t * CHUNK, 8)
        pltpu.sync_copy(src_hbm.at[pl.ds(base, CHUNK)], srcv)
        pltpu.sync_copy(dst_hbm.at[pl.ds(base, CHUNK)], dstv)
        pltpu.sync_copy(z_hbm.at[srcv], zv)
        pltpu.sync_copy(z1_hbm.at[dstv], z1v)

        lane = lax.iota(jnp.int32, L)

        def group_body(g, carry2):
            e0 = g * L
            pltpu.sync_copy(
                dfeat_hbm.at[pl.ds(pl.multiple_of(base + e0, 8), L)], dfv)

            def edge_acc(j, carry3):
                i = e0 + j
                acc = None
                for v in range(OUT // L):
                    t2 = (zv[i, pl.ds(v * L, L)] + z1v[i, pl.ds(v * L, L)]
                          + dfv[j, pl.ds(v * L, L)])
                    t3 = jnp.maximum(t2, 0.01 * t2)
                    contrib = t3 * wav[pl.ds(v * L, L)]
                    acc = contrib if acc is None else acc + contrib
                redv[pl.ds(pl.multiple_of(j * L, 8), L)] = acc
                return carry3

            lax.fori_loop(0, L, edge_acc, 0)

            ev = None
            for j in range(L):
                col = plsc.load_gather(redv, [lane * L + j])
                ev = col if ev is None else ev + col
            eegv[:] = jnp.exp(ev)

            didx = dstv[pl.ds(pl.multiple_of(e0, 8), L)]
            eevals = eegv[:]
            for l in range(L):
                plsc.addupdate_scatter(denv, [didx], eevals,
                                       mask=lane == l)

            def edge_msg(j, carry3):
                i = e0 + j
                ee = plsc.load_gather(eegv, [jnp.full((L,), j, jnp.int32)])
                for v in range(OUT // L):
                    zv[i, pl.ds(v * L, L)] = zv[i, pl.ds(v * L, L)] * ee
                return carry3

            lax.fori_loop(0, L, edge_msg, 0)
            return carry2

        lax.fori_loop(0, CHUNK // L, group_body, 0)

        pltpu.sync_copy(zv, acc_sh.at[dstv], add=True)
        return carry

    lax.fori_loop(0, NCHUNK, chunk_body, 0)

    pltpu.sync_copy(denv, den_out.at[wid])

    plsc.subcore_barrier()

    for jb in range(8):
        _fill_rows(jb)
        off = pl.multiple_of(r0 + jb * CHUNK, 8)
        pltpu.sync_copy(acc_sh.at[rowsv], zv)
        pltpu.sync_copy(zv, acc_out.at[c, pl.ds(off, CHUNK)])


def _combine_body(acc_ref, den_ref, o_ref):
    a = acc_ref[0] + acc_ref[1]
    den = jnp.sum(den_ref[...], axis=1)[:, None]
    o_ref[...] = jnp.where(den > 0, a / den, 0.0)


def _combine(acc, den, blk=400):
    return pl.pallas_call(
        _combine_body,
        grid=(N_W // blk,),
        in_specs=[pl.BlockSpec((NC, blk, OUT), lambda i: (0, i, 0)),
                  pl.BlockSpec((blk, NTILE), lambda i: (i, 0))],
        out_specs=pl.BlockSpec((blk, OUT), lambda i: (i, 0)),
        out_shape=jax.ShapeDtypeStruct((N_W, OUT), jnp.float32),
    )(acc, den.T)


def kernel(h, o, tfidf, edge_src, edge_dst, W_fc, W_fc1, W_feat, b_feat,
           W_attn):
    z = _matmul(h, W_fc, 400)
    z1 = _matmul(o, W_fc1, 400)
    dfeat = _matmul_bias(tfidf, W_feat, b_feat, 2000)
    zacc = jnp.zeros((CHUNK, OUT), jnp.float32)
    wattn = W_attn[:, 0]
    acc, den = _edge_kernel(z, z1, dfeat, edge_src, edge_dst, wattn, zacc)
    return _combine(acc, den)


# speedup vs baseline: 6.8116x; 1.5183x over previous
import functools

import jax
import jax.numpy as jnp
from jax import lax
from jax.experimental import pallas as pl
from jax.experimental.pallas import tpu as pltpu
from jax.experimental.pallas import tpu_sc as plsc

N_S = 10000
N_W = 10000
E = 320000
OUT = 128
L = 16
NC = 2
NSUB = 16
NTILE = NC * NSUB
EDGES_PER_TILE = E // NTILE
CHUNK = 80
NCHUNK = EDGES_PER_TILE // CHUNK
ROWS_PER_TILE = 640
ROW_LAST = N_W - ROWS_PER_TILE


def _mm_body(x_ref, w_ref, o_ref):
    o_ref[...] = jnp.dot(x_ref[...], w_ref[...],
                         preferred_element_type=jnp.float32)


def _matmul(x, w, blk):
    m, k = x.shape
    _, n = w.shape
    return pl.pallas_call(
        _mm_body,
        grid=(m // blk,),
        in_specs=[pl.BlockSpec((blk, k), lambda i: (i, 0)),
                  pl.BlockSpec((k, n), lambda i: (0, 0))],
        out_specs=pl.BlockSpec((blk, n), lambda i: (i, 0)),
        out_shape=jax.ShapeDtypeStruct((m, n), jnp.float32),
    )(x, w)


def _mm_bias_body(x_ref, w_ref, b_ref, o_ref):
    o_ref[...] = jnp.dot(x_ref[...], w_ref[...],
                         preferred_element_type=jnp.float32) + b_ref[...]


def _matmul_bias(x, w, b, blk):
    m, k = x.shape
    _, n = w.shape
    return pl.pallas_call(
        _mm_bias_body,
        grid=(m // blk,),
        in_specs=[pl.BlockSpec((blk, k), lambda i: (i, 0)),
                  pl.BlockSpec((k, n), lambda i: (0, 0)),
                  pl.BlockSpec((1, n), lambda i: (0, 0))],
        out_specs=pl.BlockSpec((blk, n), lambda i: (i, 0)),
        out_shape=jax.ShapeDtypeStruct((m, n), jnp.float32),
    )(x, w, b.reshape(1, n))


_MESH = plsc.VectorSubcoreMesh(core_axis_name="c", subcore_axis_name="s")


@functools.partial(
    pl.kernel,
    out_type=(jax.ShapeDtypeStruct((NC, N_W, OUT), jnp.float32),
              jax.ShapeDtypeStruct((NTILE, N_W), jnp.float32)),
    mesh=_MESH,
    compiler_params=pltpu.CompilerParams(needs_layout_passes=False),
    scratch_types=[
        pltpu.VMEM((CHUNK,), jnp.int32),
        pltpu.VMEM((CHUNK,), jnp.int32),
        pltpu.VMEM((CHUNK, OUT), jnp.float32),
        pltpu.VMEM((CHUNK, OUT), jnp.float32),
        pltpu.VMEM((CHUNK, OUT), jnp.float32),
        pltpu.VMEM((N_W,), jnp.float32),
        pltpu.VMEM((L * L,), jnp.float32),
        pltpu.VMEM((L,), jnp.float32),
        pltpu.VMEM((OUT,), jnp.float32),
        pltpu.VMEM((CHUNK,), jnp.int32),
        pltpu.SemaphoreType.DMA,
        pltpu.SemaphoreType.DMA,
        pltpu.SemaphoreType.DMA,
        pltpu.SemaphoreType.DMA,
        pltpu.SemaphoreType.DMA,
        pltpu.VMEM_SHARED((N_W, OUT), jnp.float32),
    ],
)
def _edge_kernel(z_hbm, z1_hbm, dfeat_hbm, src_hbm, dst_hbm, wattn_hbm,
                 zacc_hbm, acc_out, den_out,
                 srcv, dstv, zv, z1v, dfv, denv, redv, eegv, wav, rowsv,
                 sem_a, sem_b, sem_c, sem_d, sem_e, acc_sh):
    c = lax.axis_index("c")
    s = lax.axis_index("s")
    wid = s * NC + c
    r0 = pl.multiple_of(jnp.minimum(s * ROWS_PER_TILE, ROW_LAST), 8)

    pltpu.sync_copy(wattn_hbm, wav)

    lane16 = lax.iota(jnp.int32, L)

    def _fill_rows(jb):
        for q in range(CHUNK // L):
            rowsv[pl.ds(q * L, L)] = r0 + jb * CHUNK + q * L + lane16

    pltpu.sync_copy(zacc_hbm, zv)
    for jb in range(8):
        _fill_rows(jb)
        pltpu.sync_copy(zv, acc_sh.at[rowsv])

    zero16 = jnp.zeros((L,), jnp.float32)

    def zinit(q, carry):
        denv[pl.ds(pl.multiple_of(q * L, 8), L)] = zero16
        return carry

    lax.fori_loop(0, N_W // L, zinit, 0)
    plsc.subcore_barrier()

    base0 = wid * EDGES_PER_TILE

    def chunk_body(t, carry):
        base = pl.multiple_of(base0 + t * CHUNK, 8)
        cp_src = pltpu.async_copy(src_hbm.at[pl.ds(base, CHUNK)], srcv,
                                  sem_a)
        cp_dst = pltpu.async_copy(dst_hbm.at[pl.ds(base, CHUNK)], dstv,
                                  sem_b)
        cp_df = pltpu.async_copy(dfeat_hbm.at[pl.ds(base, CHUNK)], dfv,
                                 sem_c)
        cp_src.wait()
        cp_dst.wait()
        g_z = pltpu.async_copy(z_hbm.at[srcv], zv, sem_d)
        g_z1 = pltpu.async_copy(z1_hbm.at[dstv], z1v, sem_e)
        cp_df.wait()
        g_z.wait()
        g_z1.wait()

        lane = lax.iota(jnp.int32, L)

        def group_body(g, carry2):
            e0 = g * L

            def edge_acc(j, carry3):
                i = e0 + j
                acc = None
                for v in range(OUT // L):
                    t2 = (zv[i, pl.ds(v * L, L)] + z1v[i, pl.ds(v * L, L)]
                          + dfv[i, pl.ds(v * L, L)])
                    t3 = jnp.maximum(t2, 0.01 * t2)
                    contrib = t3 * wav[pl.ds(v * L, L)]
                    acc = contrib if acc is None else acc + contrib
                redv[pl.ds(pl.multiple_of(j * L, 8), L)] = acc
                return carry3

            lax.fori_loop(0, L, edge_acc, 0)

            ev = None
            for j in range(L):
                col = plsc.load_gather(redv, [lane * L + j])
                ev = col if ev is None else ev + col
            eegv[:] = jnp.exp(ev)

            didx = dstv[pl.ds(pl.multiple_of(e0, 8), L)]
            eevals = eegv[:]
            for l in range(L):
                plsc.addupdate_scatter(denv, [didx], eevals,
                                       mask=lane == l)

            def edge_msg(j, carry3):
                i = e0 + j
                ee = plsc.load_gather(eegv, [jnp.full((L,), j, jnp.int32)])
                for v in range(OUT // L):
                    zv[i, pl.ds(v * L, L)] = zv[i, pl.ds(v * L, L)] * ee
                return carry3

            lax.fori_loop(0, L, edge_msg, 0)
            return carry2

        lax.fori_loop(0, CHUNK // L, group_body, 0)

        pltpu.sync_copy(zv, acc_sh.at[dstv], add=True)
        return carry

    lax.fori_loop(0, NCHUNK, chunk_body, 0)

    pltpu.sync_copy(denv, den_out.at[wid])

    plsc.subcore_barrier()

    for jb in range(8):
        _fill_rows(jb)
        off = pl.multiple_of(r0 + jb * CHUNK, 8)
        pltpu.sync_copy(acc_sh.at[rowsv], zv)
        pltpu.sync_copy(zv, acc_out.at[c, pl.ds(off, CHUNK)])


def _combine_body(acc_ref, den_ref, o_ref):
    a = acc_ref[0] + acc_ref[1]
    den = jnp.sum(den_ref[...], axis=1)[:, None]
    o_ref[...] = jnp.where(den > 0, a / den, 0.0)


def _combine(acc, den, blk=400):
    return pl.pallas_call(
        _combine_body,
        grid=(N_W // blk,),
        in_specs=[pl.BlockSpec((NC, blk, OUT), lambda i: (0, i, 0)),
                  pl.BlockSpec((blk, NTILE), lambda i: (i, 0))],
        out_specs=pl.BlockSpec((blk, OUT), lambda i: (i, 0)),
        out_shape=jax.ShapeDtypeStruct((N_W, OUT), jnp.float32),
    )(acc, den.T)


def kernel(h, o, tfidf, edge_src, edge_dst, W_fc, W_fc1, W_feat, b_feat,
           W_attn):
    z = _matmul(h, W_fc, 400)
    z1 = _matmul(o, W_fc1, 400)
    dfeat = _matmul_bias(tfidf, W_feat, b_feat, 2000)
    zacc = jnp.zeros((CHUNK, OUT), jnp.float32)
    wattn = W_attn[:, 0]
    acc, den = _edge_kernel(z, z1, dfeat, edge_src, edge_dst, wattn, zacc)
    return _combine(acc, den)


# speedup vs baseline: 6.9606x; 1.0219x over previous
import functools

import jax
import jax.numpy as jnp
from jax import lax
from jax.experimental import pallas as pl
from jax.experimental.pallas import tpu as pltpu
from jax.experimental.pallas import tpu_sc as plsc

N_S = 10000
N_W = 10000
E = 320000
OUT = 128
L = 16
NC = 2
NSUB = 16
NTILE = NC * NSUB
EDGES_PER_TILE = E // NTILE
CHUNK = 80
NCHUNK = EDGES_PER_TILE // CHUNK
ROWS_PER_TILE = 640
ROW_LAST = N_W - ROWS_PER_TILE


def _mm_body(x_ref, w_ref, o_ref):
    o_ref[...] = jnp.dot(x_ref[...], w_ref[...],
                         preferred_element_type=jnp.float32)


def _matmul(x, w, blk):
    m, k = x.shape
    _, n = w.shape
    return pl.pallas_call(
        _mm_body,
        grid=(m // blk,),
        in_specs=[pl.BlockSpec((blk, k), lambda i: (i, 0)),
                  pl.BlockSpec((k, n), lambda i: (0, 0))],
        out_specs=pl.BlockSpec((blk, n), lambda i: (i, 0)),
        out_shape=jax.ShapeDtypeStruct((m, n), jnp.float32),
    )(x, w)


def _mm_bias_body(x_ref, w_ref, b_ref, o_ref):
    o_ref[...] = jnp.dot(x_ref[...], w_ref[...],
                         preferred_element_type=jnp.float32) + b_ref[...]


def _matmul_bias(x, w, b, blk):
    m, k = x.shape
    _, n = w.shape
    return pl.pallas_call(
        _mm_bias_body,
        grid=(m // blk,),
        in_specs=[pl.BlockSpec((blk, k), lambda i: (i, 0)),
                  pl.BlockSpec((k, n), lambda i: (0, 0)),
                  pl.BlockSpec((1, n), lambda i: (0, 0))],
        out_specs=pl.BlockSpec((blk, n), lambda i: (i, 0)),
        out_shape=jax.ShapeDtypeStruct((m, n), jnp.float32),
    )(x, w, b.reshape(1, n))


_MESH = plsc.VectorSubcoreMesh(core_axis_name="c", subcore_axis_name="s")


@functools.partial(
    pl.kernel,
    out_type=(jax.ShapeDtypeStruct((NC, N_W, OUT), jnp.float32),
              jax.ShapeDtypeStruct((NTILE, N_W), jnp.float32)),
    mesh=_MESH,
    compiler_params=pltpu.CompilerParams(needs_layout_passes=False),
    scratch_types=[
        pltpu.VMEM((CHUNK,), jnp.int32),
        pltpu.VMEM((CHUNK,), jnp.int32),
        pltpu.VMEM((CHUNK, OUT), jnp.float32),
        pltpu.VMEM((CHUNK, OUT), jnp.float32),
        pltpu.VMEM((CHUNK, OUT), jnp.float32),
        pltpu.VMEM((N_W,), jnp.float32),
        pltpu.VMEM((L * L,), jnp.float32),
        pltpu.VMEM((L,), jnp.float32),
        pltpu.VMEM((OUT,), jnp.float32),
        pltpu.VMEM((CHUNK,), jnp.int32),
        pltpu.SemaphoreType.DMA,
        pltpu.SemaphoreType.DMA,
        pltpu.SemaphoreType.DMA,
        pltpu.SemaphoreType.DMA,
        pltpu.SemaphoreType.DMA,
        pltpu.VMEM_SHARED((N_W, OUT), jnp.float32),
    ],
)
def _edge_kernel(z_hbm, z1_hbm, dfeat_hbm, src_hbm, dst_hbm, wattn_hbm,
                 zacc_hbm, acc_out, den_out,
                 srcv, dstv, zv, z1v, dfv, denv, redv, eegv, wav, rowsv,
                 sem_a, sem_b, sem_c, sem_d, sem_e, acc_sh):
    c = lax.axis_index("c")
    s = lax.axis_index("s")
    wid = s * NC + c
    r0 = pl.multiple_of(jnp.minimum(s * ROWS_PER_TILE, ROW_LAST), 8)

    pltpu.sync_copy(wattn_hbm, wav)

    lane16 = lax.iota(jnp.int32, L)

    def _fill_rows(jb):
        for q in range(CHUNK // L):
            rowsv[pl.ds(q * L, L)] = r0 + jb * CHUNK + q * L + lane16

    pltpu.sync_copy(zacc_hbm, zv)
    for jb in range(8):
        _fill_rows(jb)
        pltpu.sync_copy(zv, acc_sh.at[rowsv])

    zero16 = jnp.zeros((L,), jnp.float32)

    def zinit(q, carry):
        denv[pl.ds(pl.multiple_of(q * L, 8), L)] = zero16
        return carry

    lax.fori_loop(0, N_W // L, zinit, 0)
    plsc.subcore_barrier()

    base0 = wid * EDGES_PER_TILE
    was = [wav[pl.ds(v * L, L)] for v in range(OUT // L)]

    def chunk_body(t, carry):
        base = pl.multiple_of(base0 + t * CHUNK, 8)
        cp_src = pltpu.async_copy(src_hbm.at[pl.ds(base, CHUNK)], srcv,
                                  sem_a)
        cp_dst = pltpu.async_copy(dst_hbm.at[pl.ds(base, CHUNK)], dstv,
                                  sem_b)
        cp_df = pltpu.async_copy(dfeat_hbm.at[pl.ds(base, CHUNK)], dfv,
                                 sem_c)
        cp_src.wait()
        cp_dst.wait()
        g_z = pltpu.async_copy(z_hbm.at[srcv], zv, sem_d)
        g_z1 = pltpu.async_copy(z1_hbm.at[dstv], z1v, sem_e)
        cp_df.wait()
        g_z.wait()
        g_z1.wait()

        lane = lax.iota(jnp.int32, L)

        def group_body(g, carry2):
            e0 = g * L

            def edge_acc(j, carry3):
                i = e0 + j
                acc = None
                for v in range(OUT // L):
                    t2 = (zv[i, pl.ds(v * L, L)] + z1v[i, pl.ds(v * L, L)]
                          + dfv[i, pl.ds(v * L, L)])
                    t3 = jnp.maximum(t2, 0.01 * t2)
                    contrib = t3 * was[v]
                    acc = contrib if acc is None else acc + contrib
                redv[pl.ds(pl.multiple_of(j * L, 8), L)] = acc
                return carry3

            lax.fori_loop(0, L, edge_acc, 0)

            ev = None
            for j in range(L):
                col = plsc.load_gather(redv, [lane * L + j])
                ev = col if ev is None else ev + col
            eegv[:] = jnp.exp(ev)

            didx = dstv[pl.ds(pl.multiple_of(e0, 8), L)]
            eevals = eegv[:]
            for l in range(L):
                plsc.addupdate_scatter(denv, [didx], eevals,
                                       mask=lane == l)

            def edge_msg(j, carry3):
                i = e0 + j
                ee = plsc.load_gather(eegv, [jnp.full((L,), j, jnp.int32)])
                for v in range(OUT // L):
                    zv[i, pl.ds(v * L, L)] = zv[i, pl.ds(v * L, L)] * ee
                return carry3

            lax.fori_loop(0, L, edge_msg, 0)
            return carry2

        lax.fori_loop(0, CHUNK // L, group_body, 0)

        pltpu.sync_copy(zv, acc_sh.at[dstv], add=True)
        return carry

    lax.fori_loop(0, NCHUNK, chunk_body, 0)

    pltpu.sync_copy(denv, den_out.at[wid])

    plsc.subcore_barrier()

    for jb in range(8):
        _fill_rows(jb)
        off = pl.multiple_of(r0 + jb * CHUNK, 8)
        pltpu.sync_copy(acc_sh.at[rowsv], zv)
        pltpu.sync_copy(zv, acc_out.at[c, pl.ds(off, CHUNK)])


def _combine_body(acc_ref, den_ref, o_ref):
    a = acc_ref[0] + acc_ref[1]
    den = jnp.sum(den_ref[...], axis=1)[:, None]
    o_ref[...] = jnp.where(den > 0, a / den, 0.0)


def _combine(acc, den, blk=400):
    return pl.pallas_call(
        _combine_body,
        grid=(N_W // blk,),
        in_specs=[pl.BlockSpec((NC, blk, OUT), lambda i: (0, i, 0)),
                  pl.BlockSpec((blk, NTILE), lambda i: (i, 0))],
        out_specs=pl.BlockSpec((blk, OUT), lambda i: (i, 0)),
        out_shape=jax.ShapeDtypeStruct((N_W, OUT), jnp.float32),
    )(acc, den.T)


def kernel(h, o, tfidf, edge_src, edge_dst, W_fc, W_fc1, W_feat, b_feat,
           W_attn):
    z = _matmul(h, W_fc, 400)
    z1 = _matmul(o, W_fc1, 400)
    dfeat = _matmul_bias(tfidf, W_feat, b_feat, 2000)
    zacc = jnp.zeros((CHUNK, OUT), jnp.float32)
    wattn = W_attn[:, 0]
    acc, den = _edge_kernel(z, z1, dfeat, edge_src, edge_dst, wattn, zacc)
    return _combine(acc, den)
